# Initial kernel scaffold; baseline (speedup 1.0000x reference)
import functools

import jax
import jax.numpy as jnp
from jax import lax
from jax.experimental import pallas as pl
from jax.experimental.pallas import tpu as pltpu
from jax.experimental.pallas import tpu_sc as plsc

_D = 128
_NC = 2
_NS = 16
_NW = _NC * _NS
_CHUNK = 128


@functools.partial(jax.jit, static_argnames=("n",))
def _gather(idx2d, table, *, n):
    per_w = n // _NW
    nch = per_w // _CHUNK
    mesh = plsc.VectorSubcoreMesh(core_axis_name="c", subcore_axis_name="s")

    @functools.partial(
        pl.kernel,
        out_type=jax.ShapeDtypeStruct((n, _D), jnp.float32),
        mesh=mesh,
        scratch_types=[
            pltpu.VMEM((nch, _CHUNK), jnp.int32),
            pltpu.VMEM((_CHUNK, _D), jnp.float32),
            pltpu.SemaphoreType.DMA,
        ],
    )
    def body(idx_hbm, table_hbm, out_hbm, idx_v, rows_v, gsem):
        wid = lax.axis_index("s") * _NC + lax.axis_index("c")
        base = wid * per_w
        pltpu.sync_copy(idx_hbm.at[pl.ds(wid * nch, nch)], idx_v)

        @pl.loop(0, nch)
        def _(j):
            pltpu.async_copy(table_hbm.at[idx_v.at[j]], rows_v, gsem).wait()
            pltpu.sync_copy(rows_v, out_hbm.at[pl.ds(base + j * _CHUNK, _CHUNK)])

    return body(idx2d, table)


def kernel(index, table):
    b, l = index.shape
    n = b * l
    idx2d = index.reshape(n // _CHUNK, _CHUNK)
    out = _gather(idx2d, table, n=n)
    return out.reshape(b, l, _D)


# speedup vs baseline: 1.2808x; 1.2808x over previous
import functools

import jax
import jax.numpy as jnp
from jax import lax
from jax.experimental import pallas as pl
from jax.experimental.pallas import tpu as pltpu
from jax.experimental.pallas import tpu_sc as plsc

_D = 128
_NC = 2
_NS = 16
_NW = _NC * _NS
_CHUNK = 128


@functools.partial(jax.jit, static_argnames=("n",))
def _gather(idx2d, table, *, n):
    per_w = n // _NW
    nch = per_w // _CHUNK
    mesh = plsc.VectorSubcoreMesh(core_axis_name="c", subcore_axis_name="s")

    @functools.partial(
        pl.kernel,
        out_type=jax.ShapeDtypeStruct((n, _D), jnp.float32),
        mesh=mesh,
        scratch_types=[
            pltpu.VMEM((nch, _CHUNK), jnp.int32),
            pltpu.VMEM((_CHUNK, _D), jnp.float32),
            pltpu.SemaphoreType.DMA,
        ],
    )
    def body(idx_hbm, table_hbm, out_hbm, idx_v, rows_v, gsem):
        wid = lax.axis_index("s") * _NC + lax.axis_index("c")
        base = wid * per_w
        pltpu.sync_copy(idx_hbm.at[wid], idx_v)

        @pl.loop(0, nch)
        def _(j):
            pltpu.async_copy(table_hbm.at[idx_v.at[j]], rows_v, gsem).wait()
            pltpu.sync_copy(rows_v, out_hbm.at[pl.ds(base + j * _CHUNK, _CHUNK)])

    return body(idx2d, table)


def kernel(index, table):
    b, l = index.shape
    n = b * l
    idx2d = index.reshape(_NW, n // (_NW * _CHUNK), _CHUNK)
    out = _gather(idx2d, table, n=n)
    return out.reshape(b, l, _D)


# speedup vs baseline: 1.7861x; 1.3945x over previous
import functools

import jax
import jax.numpy as jnp
from jax import lax
from jax.experimental import pallas as pl
from jax.experimental.pallas import tpu as pltpu
from jax.experimental.pallas import tpu_sc as plsc

_D = 128
_NC = 2
_NS = 16
_NW = _NC * _NS
_CHUNK = 128
_NBUF = 5
_LAG = 2


@functools.partial(jax.jit, static_argnames=("n",))
def _gather(idx3d, table, *, n):
    per_w = n // _NW
    nch = per_w // _CHUNK
    ngroups = nch // _NBUF
    mesh = plsc.VectorSubcoreMesh(core_axis_name="c", subcore_axis_name="s")

    @functools.partial(
        pl.kernel,
        out_type=jax.ShapeDtypeStruct((n, _D), jnp.float32),
        mesh=mesh,
        scratch_types=[
            pltpu.VMEM((nch, _CHUNK), jnp.int32),
            pltpu.VMEM((_NBUF, _CHUNK, _D), jnp.float32),
            pltpu.SemaphoreType.DMA((_NBUF,)),
            pltpu.SemaphoreType.DMA((_NBUF,)),
        ],
    )
    def body(idx_hbm, table_hbm, out_hbm, idx_v, rows_v, gsem, ssem):
        wid = lax.axis_index("s") * _NC + lax.axis_index("c")
        base = wid * per_w
        pltpu.sync_copy(idx_hbm.at[wid], idx_v)

        def start_gather(j, b):
            pltpu.async_copy(table_hbm.at[idx_v.at[j]], rows_v.at[b],
                             gsem.at[b])

        def wait_store(i, b):
            pltpu.make_async_copy(
                rows_v.at[b],
                out_hbm.at[pl.ds(base + i * _CHUNK, _CHUNK)],
                ssem.at[b]).wait()

        def start_store(i, b):
            pltpu.make_async_copy(table_hbm.at[idx_v.at[i]], rows_v.at[b],
                                  gsem.at[b]).wait()
            pltpu.async_copy(rows_v.at[b],
                             out_hbm.at[pl.ds(base + i * _CHUNK, _CHUNK)],
                             ssem.at[b])

        def do_group(g, first):
            for b in range(_NBUF):
                j = g * _NBUF + b
                if not first:
                    wait_store(j - _NBUF, b)
                start_gather(j, b)
                if not (first and b < _LAG):
                    start_store(j - _LAG, (b - _LAG) % _NBUF)

        do_group(0, True)

        @pl.loop(1, ngroups)
        def _(g):
            do_group(g, False)

        for i in range(nch - _LAG, nch):
            start_store(i, i % _NBUF)
        for b in range(_NBUF):
            wait_store(nch - _NBUF + b, b)

    return body(idx3d, table)


def kernel(index, table):
    b, l = index.shape
    n = b * l
    idx3d = index.reshape(_NW, n // (_NW * _CHUNK), _CHUNK)
    out = _gather(idx3d, table, n=n)
    return out.reshape(b, l, _D)
